# baseline (speedup 1.0000x reference)
import functools

import jax
import jax.numpy as jnp
from jax import lax
from jax.experimental import pallas as pl
from jax.experimental.pallas import tpu as pltpu
from jax.experimental.pallas import tpu_sc as plsc

N = 10000
D = 128
HD = D // 2
E = 320000

NC = 2
NS = 16
NW = NC * NS

K = 128
CPW = 160
G = 32
NG = CPW // G
CHUNKS = NS * CPW
EPAD = CHUNKS * K
NPAD = 10240
RPT = NPAD // NS

_mesh = plsc.VectorSubcoreMesh(
    core_axis_name="c", subcore_axis_name="s", num_cores=NC, num_subcores=NS
)


def _wid(cid, sid):
    return sid * NC + cid


KD = 128
CPWD = EPAD // (NW * KD)


@functools.partial(
    pl.kernel,
    out_type=jax.ShapeDtypeStruct((NC, NPAD), jnp.float32),
    mesh=_mesh,
    scratch_types=[
        pltpu.VMEM((CPWD, KD), jnp.int32),
        pltpu.VMEM((KD,), jnp.float32),
        pltpu.VMEM_SHARED((NPAD,), jnp.float32),
    ],
)
def _deg_kernel(dst_hbm, zeros1_hbm, degp_hbm, dst_v, ones_v, degacc):
    cid = lax.axis_index("c")
    sid = lax.axis_index("s")
    wid = _wid(cid, sid)
    for i in range(KD // 16):
        ones_v[pl.ds(i * 16, 16)] = jnp.ones((16,), jnp.float32)
    pltpu.sync_copy(zeros1_hbm.at[pl.ds(sid * RPT, RPT)],
                    degacc.at[pl.ds(sid * RPT, RPT)])
    pltpu.sync_copy(dst_hbm.at[pl.ds(wid * CPWD, CPWD)], dst_v)
    plsc.subcore_barrier()

    def body(j, _):
        pltpu.sync_copy(ones_v, degacc.at[dst_v.at[j]], add=True)
        return ()

    lax.fori_loop(0, CPWD, body, ())
    plsc.subcore_barrier()
    pltpu.sync_copy(degacc.at[pl.ds(sid * RPT, RPT)],
                    degp_hbm.at[cid, pl.ds(sid * RPT, RPT)])


@functools.partial(
    pl.kernel,
    out_type=jax.ShapeDtypeStruct((NPAD, D), jnp.float32),
    mesh=_mesh,
    scratch_types=[
        pltpu.VMEM((3, G, K), jnp.int32),
        pltpu.VMEM((3, G, K), jnp.int32),
        pltpu.VMEM((4, K, HD), jnp.float32),
        pltpu.VMEM_SHARED((NPAD, HD), jnp.float32),
        pltpu.SemaphoreType.DMA((3,)),
        pltpu.SemaphoreType.DMA((4,)),
        pltpu.SemaphoreType.DMA((4,)),
    ],
    compiler_params=pltpu.CompilerParams(use_tc_tiling_on_sc=False),
)
def _acc_kernel(hs2_hbm, zeros2_hbm, src2_hbm, dst_hbm, accw_hbm,
                src_v, dst_v, rows_v, acc, isems, gsems, ssems):
    cid = lax.axis_index("c")
    sid = lax.axis_index("s")
    cbase = sid * CPW

    pltpu.sync_copy(zeros2_hbm.at[pl.ds(sid * RPT, RPT)],
                    acc.at[pl.ds(sid * RPT, RPT)])

    def load_group(g, slot):
        pltpu.async_copy(src2_hbm.at[cid, pl.ds(cbase + g * G, G)],
                         src_v.at[slot], isems.at[slot])
        pltpu.async_copy(dst_hbm.at[pl.ds(cbase + g * G, G)],
                         dst_v.at[slot], isems.at[slot])

    def wait_group(g, slot):
        pltpu.make_async_copy(src2_hbm.at[cid, pl.ds(cbase + g * G, G)],
                              src_v.at[slot], isems.at[slot]).wait()
        pltpu.make_async_copy(dst_hbm.at[pl.ds(cbase + g * G, G)],
                              dst_v.at[slot], isems.at[slot]).wait()

    for s in range(3):
        load_group(s, s)
    plsc.subcore_barrier()
    wait_group(0, 0)

    def gath(slot, r, b):
        pltpu.async_copy(hs2_hbm.at[src_v.at[slot, r]], rows_v.at[b],
                         gsems.at[b])

    def wait_gath(slot, r, b):
        pltpu.make_async_copy(hs2_hbm.at[src_v.at[slot, r]], rows_v.at[b],
                              gsems.at[b]).wait()

    def scat(slot, r, b):
        pltpu.async_copy(rows_v.at[b], acc.at[dst_v.at[slot, r]],
                         ssems.at[b], add=True)

    def wait_scat(slot, r, b):
        pltpu.make_async_copy(rows_v.at[b], acc.at[dst_v.at[slot, r]],
                              ssems.at[b]).wait()

    def prime(b):
        pltpu.async_copy(hs2_hbm.at[pl.ds(0, K)], rows_v.at[b], ssems.at[b])

    prime(2)
    prime(3)
    gath(0, 0, 0)
    gath(0, 1, 1)

    def quad(q, _):
        c0 = 4 * q
        r0 = lax.rem(c0, G)
        g = c0 // G
        slot = lax.rem(g, 3)
        nslot = lax.rem(g + 1, 3)
        crossing = r0 == G - 4

        wait_scat(slot, r0, 2)
        gath(slot, r0 + 2, 2)
        wait_gath(slot, r0, 0)
        scat(slot, r0, 0)

        wait_scat(slot, r0, 3)
        @pl.when((r0 == 0) & (g >= 1) & (g + 2 < NG))
        def _():
            load_group(g + 2, lax.rem(g + 2, 3))
        gath(slot, r0 + 3, 3)
        wait_gath(slot, r0 + 1, 1)
        scat(slot, r0 + 1, 1)

        wait_scat(slot, r0, 0)
        @pl.when(crossing & (g + 1 < NG))
        def _():
            wait_group(g + 1, nslot)
        @pl.when(c0 + 4 < CPW)
        def _():
            s2 = jnp.where(crossing, nslot, slot)
            r2 = jnp.where(crossing, 0, r0 + 4)
            gath(s2, r2, 0)
        wait_gath(slot, r0 + 2, 2)
        scat(slot, r0 + 2, 2)

        wait_scat(slot, r0, 1)
        @pl.when(c0 + 5 < CPW)
        def _():
            s2 = jnp.where(crossing, nslot, slot)
            r2 = jnp.where(crossing, 1, r0 + 5)
            gath(s2, r2, 1)
        wait_gath(slot, r0 + 3, 3)
        scat(slot, r0 + 3, 3)
        return ()

    lax.fori_loop(0, CPW // 4, quad, ())
    wait_scat(0, 0, 2)
    wait_scat(0, 0, 3)
    plsc.subcore_barrier()
    pltpu.sync_copy(acc.at[pl.ds(sid * RPT, RPT)],
                    accw_hbm.at[pl.ds(sid * RPT, RPT), pl.ds(cid * HD, HD)])


def _dinv_from(degt_ref):
    deg = 1.0 + jnp.sum(degt_ref[...], axis=1, keepdims=True)
    return lax.rsqrt(deg)


def _tc_mm_body(x_ref, w_ref, h_ref):
    h_ref[:N, :] = jnp.dot(x_ref[...], w_ref[...],
                           preferred_element_type=jnp.float32)
    h_ref[N:, :] = jnp.zeros((NPAD - N, D), jnp.float32)


def _tc_scale_body(h_ref, degt_ref, hs_ref):
    hs_ref[...] = h_ref[...] * _dinv_from(degt_ref)


def _tc2_body(accf_ref, hs1_ref, degt_ref, b_ref, w_ref, hs2_ref):
    dinv = _dinv_from(degt_ref)
    t = dinv * (accf_ref[...] + hs1_ref[...]) + b_ref[...]
    t = jnp.maximum(t, 0.0)
    hs2_ref[...] = jnp.dot(t, w_ref[...],
                           preferred_element_type=jnp.float32) * dinv


def _tc3_body(accf_ref, hs2_ref, degt_ref, b_ref, out_ref):
    dinv = _dinv_from(degt_ref)
    t = dinv * (accf_ref[...] + hs2_ref[...]) + b_ref[...]
    out_ref[...] = jnp.maximum(t, 0.0)


_tc_mm = pl.pallas_call(
    _tc_mm_body, out_shape=jax.ShapeDtypeStruct((NPAD, D), jnp.float32))

_BS = 1280
_tc_scale = pl.pallas_call(
    _tc_scale_body,
    grid=(NPAD // _BS,),
    in_specs=[
        pl.BlockSpec((_BS, D), lambda i: (i, 0)),
        pl.BlockSpec((_BS, NC), lambda i: (i, 0)),
    ],
    out_specs=pl.BlockSpec((_BS, D), lambda i: (i, 0)),
    out_shape=jax.ShapeDtypeStruct((NPAD, D), jnp.float32))

_B2 = 1280
_tc2 = pl.pallas_call(
    _tc2_body,
    grid=(NPAD // _B2,),
    in_specs=[
        pl.BlockSpec((_B2, D), lambda i: (i, 0)),
        pl.BlockSpec((_B2, D), lambda i: (i, 0)),
        pl.BlockSpec((_B2, NC), lambda i: (i, 0)),
        pl.BlockSpec((1, D), lambda i: (0, 0)),
        pl.BlockSpec((D, D), lambda i: (0, 0)),
    ],
    out_specs=pl.BlockSpec((_B2, D), lambda i: (i, 0)),
    out_shape=jax.ShapeDtypeStruct((NPAD, D), jnp.float32))

_B3 = 1000
_tc3 = pl.pallas_call(
    _tc3_body,
    grid=(N // _B3,),
    in_specs=[
        pl.BlockSpec((_B3, D), lambda i: (i, 0)),
        pl.BlockSpec((_B3, D), lambda i: (i, 0)),
        pl.BlockSpec((_B3, NC), lambda i: (i, 0)),
        pl.BlockSpec((1, D), lambda i: (0, 0)),
    ],
    out_specs=pl.BlockSpec((_B3, D), lambda i: (i, 0)),
    out_shape=jax.ShapeDtypeStruct((N, D), jnp.float32))


def _message_pass(hs, src2x, dst_p, zeros2):
    hs2 = hs.reshape(2 * NPAD, HD)
    return _acc_kernel(hs2, zeros2, src2x, dst_p)


def kernel(x, edge_index, W1, b1, W2, b2):
    src = edge_index[0]
    dst = edge_index[1]
    pad_idx = (N + jnp.arange(EPAD - E, dtype=jnp.int32) % (NPAD - N)).astype(
        jnp.int32)
    src_pad2 = jnp.concatenate([src, pad_idx]) * 2
    src2x = jnp.stack([src_pad2, src_pad2 + 1]).reshape(NC, CHUNKS, K)
    dst_pad = jnp.concatenate([dst, pad_idx])
    dst_p = dst_pad.reshape(CHUNKS, K)
    zeros1 = jnp.zeros((NPAD,), jnp.float32)
    zeros2 = jnp.zeros((NPAD, HD), jnp.float32)

    h1 = _tc_mm(x, W1)
    degp = _deg_kernel(dst_p, zeros1)
    degt = degp.T
    hs1 = _tc_scale(h1, degt)
    accf1 = _message_pass(hs1, src2x, dst_p, zeros2)
    hs2 = _tc2(accf1, hs1, degt, b1.reshape(1, D), W2)
    accf2 = _message_pass(hs2, src2x, dst_p, zeros2)
    return _tc3(accf2, hs2, degt, b2.reshape(1, D))


# speedup vs baseline: 1.0221x; 1.0221x over previous
import functools

import jax
import jax.numpy as jnp
from jax import lax
from jax.experimental import pallas as pl
from jax.experimental.pallas import tpu as pltpu
from jax.experimental.pallas import tpu_sc as plsc

N = 10000
D = 128
HD = D // 2
E = 320000

NC = 2
NS = 16
NW = NC * NS

K = 128
CPW = 160
G = 32
NG = CPW // G
CHUNKS = NS * CPW
EPAD = CHUNKS * K
NPAD = 10240
RPT = NPAD // NS

_mesh = plsc.VectorSubcoreMesh(
    core_axis_name="c", subcore_axis_name="s", num_cores=NC, num_subcores=NS
)


def _wid(cid, sid):
    return sid * NC + cid


KD = 128
CPWD = EPAD // (NW * KD)


@functools.partial(
    pl.kernel,
    out_type=jax.ShapeDtypeStruct((NC, NPAD), jnp.float32),
    mesh=_mesh,
    scratch_types=[
        pltpu.VMEM((CPWD, KD), jnp.int32),
        pltpu.VMEM((KD,), jnp.float32),
        pltpu.VMEM_SHARED((NPAD,), jnp.float32),
    ],
)
def _deg_kernel(dst_hbm, zeros1_hbm, degp_hbm, dst_v, ones_v, degacc):
    cid = lax.axis_index("c")
    sid = lax.axis_index("s")
    wid = _wid(cid, sid)
    for i in range(KD // 16):
        ones_v[pl.ds(i * 16, 16)] = jnp.ones((16,), jnp.float32)
    pltpu.sync_copy(zeros1_hbm.at[pl.ds(sid * RPT, RPT)],
                    degacc.at[pl.ds(sid * RPT, RPT)])
    pltpu.sync_copy(dst_hbm.at[pl.ds(wid * CPWD, CPWD)], dst_v)
    plsc.subcore_barrier()

    def body(j, _):
        pltpu.sync_copy(ones_v, degacc.at[dst_v.at[j]], add=True)
        return ()

    lax.fori_loop(0, CPWD, body, ())
    plsc.subcore_barrier()
    pltpu.sync_copy(degacc.at[pl.ds(sid * RPT, RPT)],
                    degp_hbm.at[cid, pl.ds(sid * RPT, RPT)])


@functools.partial(
    pl.kernel,
    out_type=jax.ShapeDtypeStruct((NPAD, D), jnp.float32),
    mesh=_mesh,
    scratch_types=[
        pltpu.VMEM((3, G, K), jnp.int32),
        pltpu.VMEM((3, G, K), jnp.int32),
        pltpu.VMEM((4, K, HD), jnp.float32),
        pltpu.VMEM_SHARED((NPAD, HD), jnp.float32),
        pltpu.SemaphoreType.DMA((3,)),
        pltpu.SemaphoreType.DMA((4,)),
        pltpu.SemaphoreType.DMA((4,)),
    ],
    compiler_params=pltpu.CompilerParams(use_tc_tiling_on_sc=False),
)
def _acc_kernel(hs2_hbm, zeros2_hbm, src2_hbm, dst_hbm, accw_hbm,
                src_v, dst_v, rows_v, acc, isems, gsems, ssems):
    cid = lax.axis_index("c")
    sid = lax.axis_index("s")
    cbase = sid * CPW

    pltpu.sync_copy(zeros2_hbm.at[pl.ds(sid * RPT, RPT)],
                    acc.at[pl.ds(sid * RPT, RPT)])

    def load_group(g, slot):
        pltpu.async_copy(src2_hbm.at[cid, pl.ds(cbase + g * G, G)],
                         src_v.at[slot], isems.at[slot])
        pltpu.async_copy(dst_hbm.at[pl.ds(cbase + g * G, G)],
                         dst_v.at[slot], isems.at[slot])

    def wait_group(g, slot):
        pltpu.make_async_copy(src2_hbm.at[cid, pl.ds(cbase + g * G, G)],
                              src_v.at[slot], isems.at[slot]).wait()
        pltpu.make_async_copy(dst_hbm.at[pl.ds(cbase + g * G, G)],
                              dst_v.at[slot], isems.at[slot]).wait()

    for s in range(3):
        load_group(s, s)
    plsc.subcore_barrier()
    wait_group(0, 0)

    def gath(slot, r, b):
        pltpu.async_copy(hs2_hbm.at[src_v.at[slot, r]], rows_v.at[b],
                         gsems.at[b])

    def wait_gath(slot, r, b):
        pltpu.make_async_copy(hs2_hbm.at[src_v.at[slot, r]], rows_v.at[b],
                              gsems.at[b]).wait()

    def scat(slot, r, b):
        pltpu.async_copy(rows_v.at[b], acc.at[dst_v.at[slot, r]],
                         ssems.at[b], add=True)

    def wait_scat(slot, r, b):
        pltpu.make_async_copy(rows_v.at[b], acc.at[dst_v.at[slot, r]],
                              ssems.at[b]).wait()

    def prime(b):
        pltpu.async_copy(hs2_hbm.at[pl.ds(0, K)], rows_v.at[b], ssems.at[b])

    prime(2)
    prime(3)
    gath(0, 0, 0)
    gath(0, 1, 1)

    def quad(q, _):
        c0 = 4 * q
        r0 = lax.rem(c0, G)
        g = c0 // G
        slot = lax.rem(g, 3)
        nslot = lax.rem(g + 1, 3)
        crossing = r0 == G - 4

        wait_scat(slot, r0, 2)
        gath(slot, r0 + 2, 2)
        wait_gath(slot, r0, 0)
        scat(slot, r0, 0)

        wait_scat(slot, r0, 3)
        @pl.when((r0 == 0) & (g >= 1) & (g + 2 < NG))
        def _():
            load_group(g + 2, lax.rem(g + 2, 3))
        gath(slot, r0 + 3, 3)
        wait_gath(slot, r0 + 1, 1)
        scat(slot, r0 + 1, 1)

        wait_scat(slot, r0, 0)
        @pl.when(crossing & (g + 1 < NG))
        def _():
            wait_group(g + 1, nslot)
        @pl.when(c0 + 4 < CPW)
        def _():
            s2 = jnp.where(crossing, nslot, slot)
            r2 = jnp.where(crossing, 0, r0 + 4)
            gath(s2, r2, 0)
        wait_gath(slot, r0 + 2, 2)
        scat(slot, r0 + 2, 2)

        wait_scat(slot, r0, 1)
        @pl.when(c0 + 5 < CPW)
        def _():
            s2 = jnp.where(crossing, nslot, slot)
            r2 = jnp.where(crossing, 1, r0 + 5)
            gath(s2, r2, 1)
        wait_gath(slot, r0 + 3, 3)
        scat(slot, r0 + 3, 3)
        return ()

    lax.fori_loop(0, CPW // 4, quad, ())
    wait_scat(0, 0, 2)
    wait_scat(0, 0, 3)
    plsc.subcore_barrier()
    pltpu.sync_copy(acc.at[pl.ds(sid * RPT, RPT)],
                    accw_hbm.at[pl.ds(sid * RPT, RPT), pl.ds(cid * HD, HD)])


def _dinv_from(degt_ref):
    deg = 1.0 + jnp.sum(degt_ref[...], axis=1, keepdims=True)
    return lax.rsqrt(deg)


def _tc_mm_body(x_ref, w_ref, h_ref):
    h_ref[:N, :] = jnp.dot(x_ref[...], w_ref[...],
                           preferred_element_type=jnp.float32)
    h_ref[N:, :] = jnp.zeros((NPAD - N, D), jnp.float32)


def _tc_scale_body(h_ref, degt_ref, hs_ref):
    hs_ref[...] = h_ref[...] * _dinv_from(degt_ref)


def _tc2_body(accf_ref, hs1_ref, degt_ref, b_ref, w_ref, hs2_ref):
    dinv = _dinv_from(degt_ref)
    t = dinv * (accf_ref[...] + hs1_ref[...]) + b_ref[...]
    t = jnp.maximum(t, 0.0)
    hs2_ref[...] = jnp.dot(t, w_ref[...],
                           preferred_element_type=jnp.float32) * dinv


def _tc3_body(accf_ref, hs2_ref, degt_ref, b_ref, out_ref):
    dinv = _dinv_from(degt_ref)
    t = dinv[:N, :] * (accf_ref[:N, :] + hs2_ref[:N, :]) + b_ref[...]
    out_ref[...] = jnp.maximum(t, 0.0)


_tc_mm = pl.pallas_call(
    _tc_mm_body, out_shape=jax.ShapeDtypeStruct((NPAD, D), jnp.float32))
_tc_scale = pl.pallas_call(
    _tc_scale_body, out_shape=jax.ShapeDtypeStruct((NPAD, D), jnp.float32))
_tc2 = pl.pallas_call(
    _tc2_body, out_shape=jax.ShapeDtypeStruct((NPAD, D), jnp.float32))
_tc3 = pl.pallas_call(
    _tc3_body, out_shape=jax.ShapeDtypeStruct((N, D), jnp.float32))


def _message_pass(hs, src2x, dst_p, zeros2):
    hs2 = hs.reshape(2 * NPAD, HD)
    return _acc_kernel(hs2, zeros2, src2x, dst_p)


def kernel(x, edge_index, W1, b1, W2, b2):
    src = edge_index[0]
    dst = edge_index[1]
    pad_idx = (N + jnp.arange(EPAD - E, dtype=jnp.int32) % (NPAD - N)).astype(
        jnp.int32)
    src_pad2 = jnp.concatenate([src, pad_idx]) * 2
    src2x = jnp.stack([src_pad2, src_pad2 + 1]).reshape(NC, CHUNKS, K)
    dst_pad = jnp.concatenate([dst, pad_idx])
    dst_p = dst_pad.reshape(CHUNKS, K)
    zeros1 = jnp.zeros((NPAD,), jnp.float32)
    zeros2 = jnp.zeros((NPAD, HD), jnp.float32)

    h1 = _tc_mm(x, W1)
    degp = _deg_kernel(dst_p, zeros1)
    degt = degp.T
    hs1 = _tc_scale(h1, degt)
    accf1 = _message_pass(hs1, src2x, dst_p, zeros2)
    hs2 = _tc2(accf1, hs1, degt, b1.reshape(1, D), W2)
    accf2 = _message_pass(hs2, src2x, dst_p, zeros2)
    return _tc3(accf2, hs2, degt, b2.reshape(1, D))
